# baseline (speedup 1.0000x reference)
import math

import numpy as np
import jax
import jax.numpy as jnp
from jax.experimental import pallas as pl
from jax.experimental.pallas import tpu as pltpu

_L = 2048
_H = 12
_D = 64
_SAMPLE = 40
_TOPU = 40
_SCALE = 1.0 / math.sqrt(_D)

_RB = 256
_NRB = _L // _RB
_CB = 128
_SUB = _L // 8


def _count_matrix():
    idx = jax.random.randint(jax.random.key(42), (_L, _SAMPLE), 0, _L)
    c = jnp.zeros((_L, _L), jnp.float32)
    return c.at[jnp.arange(_L)[:, None], idx].add(1.0)


def _m_kernel(q_ref, k_ref, c_ref, m_ref):
    q = q_ref[0]
    k = k_ref[0]
    c = c_ref[...]
    s = jax.lax.dot_general(q, k, (((1,), (1,)), ((), ())),
                            preferred_element_type=jnp.float32)
    smax = jnp.max(jnp.where(c > 0.0, s, -jnp.inf), axis=1, keepdims=True)
    ssum = jnp.sum(s * c, axis=1, keepdims=True)
    m_ref[0] = smax - ssum * (1.0 / _L)


def _attn_kernel(m_ref, q_ref, k_ref, v_ref, o_ref, mtop_ref):
    m2 = m_ref[0]
    gi = (jax.lax.broadcasted_iota(jnp.int32, (8, _SUB), 0) * _SUB
          + jax.lax.broadcasted_iota(jnp.int32, (8, _SUB), 1))

    def body(u, mm):
        mx = jnp.max(mm)
        idx = jnp.min(jnp.where(mm == mx, gi, _L))
        mtop_ref[pl.ds(u, 1), :] = idx.astype(jnp.float32)[None, None]
        return jnp.where(gi == idx, -jnp.inf, mm)

    jax.lax.fori_loop(0, _TOPU, body, m2)

    q = q_ref[0]
    k = k_ref[0]
    v = v_ref[0]
    mtop = mtop_ref[...]
    lane = jax.lax.broadcasted_iota(
        jnp.int32, (_TOPU, _L), 1).astype(jnp.float32)
    oh = (lane == mtop).astype(jnp.float32)

    q_red = jnp.dot(oh, q, preferred_element_type=jnp.float32)
    s = jax.lax.dot_general(q_red, k, (((1,), (1,)), ((), ())),
                            preferred_element_type=jnp.float32) * _SCALE
    s = jnp.where(lane > mtop, -jnp.inf, s)
    smx = jnp.max(s, axis=1, keepdims=True)
    p = jnp.exp(s - smx)
    attn = p / jnp.sum(p, axis=1, keepdims=True)
    upd = jnp.dot(attn, v, preferred_element_type=jnp.float32)

    tril = (jax.lax.broadcasted_iota(jnp.int32, (_CB, _CB), 0)
            >= jax.lax.broadcasted_iota(jnp.int32, (_CB, _CB), 1)
            ).astype(jnp.float32)
    blocks = []
    carry = jnp.zeros((1, _D), jnp.float32)
    for b in range(_L // _CB):
        blk = v[b * _CB:(b + 1) * _CB]
        blocks.append(
            jnp.dot(tril, blk, preferred_element_type=jnp.float32) + carry)
        carry = carry + jnp.sum(blk, axis=0, keepdims=True)
    ctx = jnp.concatenate(blocks, axis=0)

    scat = jax.lax.dot_general(oh, upd, (((0,), (0,)), ((), ())),
                               preferred_element_type=jnp.float32)
    selc = jax.lax.dot_general(oh, jnp.ones((_TOPU, _D), jnp.float32),
                               (((0,), (0,)), ((), ())),
                               preferred_element_type=jnp.float32)
    o_ref[0] = jnp.where(selc > 0.5, scat, ctx)


def kernel(queries, keys, values):
    assert queries.shape == (1, _L, _H, _D), queries.shape
    qh = jnp.transpose(queries[0], (1, 0, 2))
    kh = jnp.transpose(keys[0], (1, 0, 2))
    vh = jnp.transpose(values[0], (1, 0, 2))
    c = _count_matrix()

    m3 = pl.pallas_call(
        _m_kernel,
        grid=(_NRB, _H),
        in_specs=[
            pl.BlockSpec((1, _RB, _D), lambda rb, h: (h, rb, 0)),
            pl.BlockSpec((1, _L, _D), lambda rb, h: (h, 0, 0)),
            pl.BlockSpec((_RB, _L), lambda rb, h: (rb, 0)),
        ],
        out_specs=pl.BlockSpec((1, _RB, 1), lambda rb, h: (h * _NRB + rb, 0, 0)),
        out_shape=jax.ShapeDtypeStruct((_H * _NRB, _RB, 1), jnp.float32),
    )(qh, kh, c)

    m = m3.reshape(_H, _L).reshape(_H, 8, _SUB)

    ctx = pl.pallas_call(
        _attn_kernel,
        grid=(_H,),
        in_specs=[
            pl.BlockSpec((1, 8, _SUB), lambda h: (h, 0, 0)),
            pl.BlockSpec((1, _L, _D), lambda h: (h, 0, 0)),
            pl.BlockSpec((1, _L, _D), lambda h: (h, 0, 0)),
            pl.BlockSpec((1, _L, _D), lambda h: (h, 0, 0)),
        ],
        out_specs=pl.BlockSpec((1, _L, _D), lambda h: (h, 0, 0)),
        out_shape=jax.ShapeDtypeStruct((_H, _L, _D), jnp.float32),
        scratch_shapes=[pltpu.VMEM((_TOPU, 1), jnp.float32)],
    )(m, qh, kh, vh)

    return jnp.transpose(ctx, (1, 0, 2))[None]


# speedup vs baseline: 1.6398x; 1.6398x over previous
import math

import numpy as np
import jax
import jax.numpy as jnp
from jax.experimental import pallas as pl
from jax.experimental.pallas import tpu as pltpu

_L = 2048
_H = 12
_D = 64
_SAMPLE = 40
_TOPU = 40
_SCALE = 1.0 / math.sqrt(_D)

_RB = 256
_NRB = _L // _RB
_CB = 128
_SUB = _L // 8


def _tf2x32(k1, k2, x1, x2):
    rot1 = (13, 15, 26, 6)
    rot2 = (17, 29, 16, 24)
    ks = [np.uint32(k1), np.uint32(k2),
          np.uint32(k1) ^ np.uint32(k2) ^ np.uint32(0x1BD11BDA)]
    a = (x1 + ks[0]).astype(np.uint32)
    b = (x2 + ks[1]).astype(np.uint32)

    def rol(v, r):
        return ((v << np.uint32(r)) | (v >> np.uint32(32 - r))).astype(np.uint32)

    def rounds(a, b, rots):
        for r in rots:
            a = (a + b).astype(np.uint32)
            b = a ^ rol(b, r)
        return a, b

    for i, rots in enumerate((rot1, rot2, rot1, rot2, rot1)):
        a, b = rounds(a, b, rots)
        a = (a + ks[(i + 1) % 3]).astype(np.uint32)
        b = (b + ks[(i + 2) % 3] + np.uint32(i + 1)).astype(np.uint32)
    return a, b


def _sample_indices():
    b1, b2 = _tf2x32(np.uint32(0), np.uint32(42),
                     np.zeros(2, np.uint32), np.arange(2, dtype=np.uint32))
    ka, kb = (b1[0], b2[0]), (b1[1], b2[1])
    n = _L * _SAMPLE

    def bits(key):
        o1, o2 = _tf2x32(key[0], key[1],
                         np.zeros(n, np.uint32), np.arange(n, dtype=np.uint32))
        return o1 ^ o2

    higher, lower = bits(ka), bits(kb)
    span = np.uint32(_L)
    mult = np.uint32((2 ** 16) % _L)
    mult = np.uint32((mult * mult) % span)
    off = ((higher % span) * mult + (lower % span)).astype(np.uint32) % span
    return off.reshape(_L, _SAMPLE).astype(np.int64)


def _count_matrix_np():
    idx = _sample_indices()
    c = np.zeros((_L, _L), np.float32)
    np.add.at(c, (np.arange(_L)[:, None], idx), 1.0)
    return c


_C = _count_matrix_np()


def _m_kernel(q_ref, k_ref, c_ref, m_ref):
    q = q_ref[0]
    k = k_ref[0]
    c = c_ref[...]
    s = jax.lax.dot_general(q, k, (((1,), (1,)), ((), ())),
                            preferred_element_type=jnp.float32)
    smax = jnp.max(jnp.where(c > 0.0, s, -jnp.inf), axis=1, keepdims=True)
    ssum = jnp.sum(s * c, axis=1, keepdims=True)
    m_ref[0] = smax - ssum * (1.0 / _L)


def _attn_kernel(m_ref, q_ref, k_ref, v_ref, o_ref, mtop_ref):
    m2 = m_ref[0]
    gi = (jax.lax.broadcasted_iota(jnp.int32, (8, _SUB), 0) * _SUB
          + jax.lax.broadcasted_iota(jnp.int32, (8, _SUB), 1))

    def body(u, mm):
        mx = jnp.max(mm)
        idx = jnp.min(jnp.where(mm == mx, gi, _L))
        mtop_ref[pl.ds(u, 1), :] = idx.astype(jnp.float32)[None, None]
        return jnp.where(gi == idx, -jnp.inf, mm)

    jax.lax.fori_loop(0, _TOPU, body, m2)

    q = q_ref[0]
    k = k_ref[0]
    v = v_ref[0]
    mtop = mtop_ref[...]
    lane = jax.lax.broadcasted_iota(
        jnp.int32, (_TOPU, _L), 1).astype(jnp.float32)
    oh = (lane == mtop).astype(jnp.float32)

    q_red = jnp.dot(oh, q, preferred_element_type=jnp.float32)
    s = jax.lax.dot_general(q_red, k, (((1,), (1,)), ((), ())),
                            preferred_element_type=jnp.float32) * _SCALE
    s = jnp.where(lane > mtop, -jnp.inf, s)
    smx = jnp.max(s, axis=1, keepdims=True)
    p = jnp.exp(s - smx)
    attn = p / jnp.sum(p, axis=1, keepdims=True)
    upd = jnp.dot(attn, v, preferred_element_type=jnp.float32)

    tril = (jax.lax.broadcasted_iota(jnp.int32, (_CB, _CB), 0)
            >= jax.lax.broadcasted_iota(jnp.int32, (_CB, _CB), 1)
            ).astype(jnp.float32)
    blocks = []
    carry = jnp.zeros((1, _D), jnp.float32)
    for b in range(_L // _CB):
        blk = v[b * _CB:(b + 1) * _CB]
        blocks.append(
            jnp.dot(tril, blk, preferred_element_type=jnp.float32) + carry)
        carry = carry + jnp.sum(blk, axis=0, keepdims=True)
    ctx = jnp.concatenate(blocks, axis=0)

    scat = jax.lax.dot_general(oh, upd, (((0,), (0,)), ((), ())),
                               preferred_element_type=jnp.float32)
    selc = jax.lax.dot_general(oh, jnp.ones((_TOPU, _D), jnp.float32),
                               (((0,), (0,)), ((), ())),
                               preferred_element_type=jnp.float32)
    o_ref[0] = jnp.where(selc > 0.5, scat, ctx)


def kernel(queries, keys, values):
    assert queries.shape == (1, _L, _H, _D), queries.shape
    qh = jnp.transpose(queries[0], (1, 0, 2))
    kh = jnp.transpose(keys[0], (1, 0, 2))
    vh = jnp.transpose(values[0], (1, 0, 2))
    c = jnp.asarray(_C)

    m3 = pl.pallas_call(
        _m_kernel,
        grid=(_NRB, _H),
        in_specs=[
            pl.BlockSpec((1, _RB, _D), lambda rb, h: (h, rb, 0)),
            pl.BlockSpec((1, _L, _D), lambda rb, h: (h, 0, 0)),
            pl.BlockSpec((_RB, _L), lambda rb, h: (rb, 0)),
        ],
        out_specs=pl.BlockSpec((1, _RB, 1), lambda rb, h: (h * _NRB + rb, 0, 0)),
        out_shape=jax.ShapeDtypeStruct((_H * _NRB, _RB, 1), jnp.float32),
    )(qh, kh, c)

    m = m3.reshape(_H, _L).reshape(_H, 8, _SUB)

    ctx = pl.pallas_call(
        _attn_kernel,
        grid=(_H,),
        in_specs=[
            pl.BlockSpec((1, 8, _SUB), lambda h: (h, 0, 0)),
            pl.BlockSpec((1, _L, _D), lambda h: (h, 0, 0)),
            pl.BlockSpec((1, _L, _D), lambda h: (h, 0, 0)),
            pl.BlockSpec((1, _L, _D), lambda h: (h, 0, 0)),
        ],
        out_specs=pl.BlockSpec((1, _L, _D), lambda h: (h, 0, 0)),
        out_shape=jax.ShapeDtypeStruct((_H, _L, _D), jnp.float32),
        scratch_shapes=[pltpu.VMEM((_TOPU, 1), jnp.float32)],
    )(m, qh, kh, vh)

    return jnp.transpose(ctx, (1, 0, 2))[None]


# speedup vs baseline: 2.7452x; 1.6740x over previous
import math

import numpy as np
import jax
import jax.numpy as jnp
from jax.experimental import pallas as pl
from jax.experimental.pallas import tpu as pltpu

_L = 2048
_H = 12
_D = 64
_SAMPLE = 40
_TOPU = 40
_SCALE = 1.0 / math.sqrt(_D)

_RB = 256
_NRB = _L // _RB
_CB = 128
_SUB = _L // 8


def _tf2x32(k1, k2, x1, x2):
    rot1 = (13, 15, 26, 6)
    rot2 = (17, 29, 16, 24)
    ks = [np.uint32(k1), np.uint32(k2),
          np.uint32(k1) ^ np.uint32(k2) ^ np.uint32(0x1BD11BDA)]
    a = (x1 + ks[0]).astype(np.uint32)
    b = (x2 + ks[1]).astype(np.uint32)

    def rol(v, r):
        return ((v << np.uint32(r)) | (v >> np.uint32(32 - r))).astype(np.uint32)

    def rounds(a, b, rots):
        for r in rots:
            a = (a + b).astype(np.uint32)
            b = a ^ rol(b, r)
        return a, b

    for i, rots in enumerate((rot1, rot2, rot1, rot2, rot1)):
        a, b = rounds(a, b, rots)
        a = (a + ks[(i + 1) % 3]).astype(np.uint32)
        b = (b + ks[(i + 2) % 3] + np.uint32(i + 1)).astype(np.uint32)
    return a, b


def _sample_indices():
    b1, b2 = _tf2x32(np.uint32(0), np.uint32(42),
                     np.zeros(2, np.uint32), np.arange(2, dtype=np.uint32))
    ka, kb = (b1[0], b2[0]), (b1[1], b2[1])
    n = _L * _SAMPLE

    def bits(key):
        o1, o2 = _tf2x32(key[0], key[1],
                         np.zeros(n, np.uint32), np.arange(n, dtype=np.uint32))
        return o1 ^ o2

    higher, lower = bits(ka), bits(kb)
    span = np.uint32(_L)
    mult = np.uint32((2 ** 16) % _L)
    mult = np.uint32((mult * mult) % span)
    off = ((higher % span) * mult + (lower % span)).astype(np.uint32) % span
    return off.reshape(_L, _SAMPLE).astype(np.int64)


def _count_matrix_np():
    idx = _sample_indices()
    c = np.zeros((_L, _L), np.float32)
    np.add.at(c, (np.arange(_L)[:, None], idx), 1.0)
    return c


_C = _count_matrix_np()


def _m_kernel(q_ref, k_ref, c_ref, m_ref):
    q = q_ref[0]
    k = k_ref[0]
    c = c_ref[...]
    s = jax.lax.dot_general(q, k, (((1,), (1,)), ((), ())),
                            preferred_element_type=jnp.float32)
    smax = jnp.max(jnp.where(c > 0.0, s, -jnp.inf), axis=1, keepdims=True)
    ssum = jnp.sum(s * c, axis=1, keepdims=True)
    m_ref[0] = smax - ssum * (1.0 / _L)


def _topk_kernel(m_ref, mt_ref):
    m3 = m_ref[...]
    gi = (jax.lax.broadcasted_iota(jnp.int32, (_H, 8, _SUB), 1) * _SUB
          + jax.lax.broadcasted_iota(jnp.int32, (_H, 8, _SUB), 2))

    def body(u, mm):
        mx = jnp.max(jnp.max(mm, axis=2, keepdims=True), axis=1,
                     keepdims=True)
        cand = jnp.where(mm == mx, gi, _L)
        idx = jnp.min(jnp.min(cand, axis=2, keepdims=True), axis=1,
                      keepdims=True)
        mt_ref[:, pl.ds(u, 1), :] = idx.astype(jnp.float32)
        return jnp.where(gi == idx, -jnp.inf, mm)

    jax.lax.fori_loop(0, _TOPU, body, m3)


def _attn_kernel(mt_ref, q_ref, k_ref, v_ref, o_ref):
    q = q_ref[0]
    k = k_ref[0]
    v = v_ref[0]
    mtop = mt_ref[0]
    lane = jax.lax.broadcasted_iota(
        jnp.int32, (_TOPU, _L), 1).astype(jnp.float32)
    oh = (lane == mtop).astype(jnp.float32)

    q_red = jnp.dot(oh, q, preferred_element_type=jnp.float32)
    s = jax.lax.dot_general(q_red, k, (((1,), (1,)), ((), ())),
                            preferred_element_type=jnp.float32) * _SCALE
    s = jnp.where(lane > mtop, -jnp.inf, s)
    smx = jnp.max(s, axis=1, keepdims=True)
    p = jnp.exp(s - smx)
    attn = p / jnp.sum(p, axis=1, keepdims=True)
    upd = jnp.dot(attn, v, preferred_element_type=jnp.float32)

    nb = _L // _CB
    tril = (jax.lax.broadcasted_iota(jnp.int32, (nb, _CB, _CB), 1)
            >= jax.lax.broadcasted_iota(jnp.int32, (nb, _CB, _CB), 2)
            ).astype(jnp.float32)
    vb = v.reshape(nb, _CB, _D)
    intra = jax.lax.dot_general(
        tril, vb, (((2,), (1,)), ((0,), (0,))),
        preferred_element_type=jnp.float32)
    bs = intra[:, _CB - 1:_CB, :].reshape(nb, _D)
    stril = (jax.lax.broadcasted_iota(jnp.int32, (nb, nb), 0)
             > jax.lax.broadcasted_iota(jnp.int32, (nb, nb), 1)
             ).astype(jnp.float32)
    off = jnp.dot(stril, bs, preferred_element_type=jnp.float32)
    ctx = (intra + off.reshape(nb, 1, _D)).reshape(_L, _D)

    scat = jax.lax.dot_general(oh, upd, (((0,), (0,)), ((), ())),
                               preferred_element_type=jnp.float32)
    selc = jax.lax.dot_general(oh, jnp.ones((_TOPU, _D), jnp.float32),
                               (((0,), (0,)), ((), ())),
                               preferred_element_type=jnp.float32)
    o_ref[0] = jnp.where(selc > 0.5, scat, ctx)


def kernel(queries, keys, values):
    assert queries.shape == (1, _L, _H, _D), queries.shape
    qh = jnp.transpose(queries[0], (1, 0, 2))
    kh = jnp.transpose(keys[0], (1, 0, 2))
    vh = jnp.transpose(values[0], (1, 0, 2))
    c = jnp.asarray(_C)

    m3 = pl.pallas_call(
        _m_kernel,
        grid=(_NRB, _H),
        in_specs=[
            pl.BlockSpec((1, _RB, _D), lambda rb, h: (h, rb, 0)),
            pl.BlockSpec((1, _L, _D), lambda rb, h: (h, 0, 0)),
            pl.BlockSpec((_RB, _L), lambda rb, h: (rb, 0)),
        ],
        out_specs=pl.BlockSpec((1, _RB, 1), lambda rb, h: (h * _NRB + rb, 0, 0)),
        out_shape=jax.ShapeDtypeStruct((_H * _NRB, _RB, 1), jnp.float32),
    )(qh, kh, c)

    m = m3.reshape(_H, _L).reshape(_H, 8, _SUB)

    mt = pl.pallas_call(
        _topk_kernel,
        grid=(1,),
        in_specs=[pl.BlockSpec((_H, 8, _SUB), lambda i: (0, 0, 0))],
        out_specs=pl.BlockSpec((_H, _TOPU, 1), lambda i: (0, 0, 0)),
        out_shape=jax.ShapeDtypeStruct((_H, _TOPU, 1), jnp.float32),
    )(m)

    ctx = pl.pallas_call(
        _attn_kernel,
        grid=(_H,),
        in_specs=[
            pl.BlockSpec((1, _TOPU, 1), lambda h: (h, 0, 0)),
            pl.BlockSpec((1, _L, _D), lambda h: (h, 0, 0)),
            pl.BlockSpec((1, _L, _D), lambda h: (h, 0, 0)),
            pl.BlockSpec((1, _L, _D), lambda h: (h, 0, 0)),
        ],
        out_specs=pl.BlockSpec((1, _L, _D), lambda h: (h, 0, 0)),
        out_shape=jax.ShapeDtypeStruct((_H, _L, _D), jnp.float32),
    )(mt, qh, kh, vh)

    return jnp.transpose(ctx, (1, 0, 2))[None]
